# baseline (speedup 1.0000x reference)
import math

import jax
import jax.numpy as jnp
from jax.experimental import pallas as pl
from jax.experimental.pallas import tpu as pltpu

B, S, IN, OUT, E, R, DK = 2, 4096, 768, 768, 8, 8, 32
SCALING = 16.0 / 8.0
TILE = 2048
KQ = E * DK

_NT = (((1,), (1,)), ((), ()))


def _kernel(x_ref, w_ref, wq_ref, wk_ref, a_ref, bm_ref, b_ref, out_ref,
            wb, wqb, wkb, ab, bmb, segb):
    @pl.when(pl.program_id(0) == 0)
    def _prep():
        wb[...] = w_ref[...].astype(jnp.bfloat16)
        wqb[...] = wq_ref[...].astype(jnp.bfloat16)
        wkb[...] = wk_ref[...].astype(jnp.bfloat16)
        ab[...] = a_ref[...].astype(jnp.bfloat16)
        bmt = jax.lax.transpose(bm_ref[...], (0, 2, 1))
        bmb[...] = (bmt.reshape(E * R, OUT) * SCALING).astype(jnp.bfloat16)
        j = jax.lax.broadcasted_iota(jnp.int32, (KQ, E * R), 0) // DK
        e = jax.lax.broadcasted_iota(jnp.int32, (KQ, E * R), 1) // R
        segb[...] = jnp.where(j == e, 1.0 / math.sqrt(DK), 0.0
                              ).astype(jnp.bfloat16)

    xb = x_ref[...].astype(jnp.bfloat16)

    dot = lambda u, v: jax.lax.dot_general(
        u, v, _NT, preferred_element_type=jnp.float32)
    result = dot(xb, wb[...])
    q = dot(xb, wqb[...])
    k = dot(xb, wkb[...])
    h = dot(xb, ab[...])

    qk = (q * k).astype(jnp.bfloat16)
    s64 = jnp.dot(qk, segb[...], preferred_element_type=jnp.float32)
    m = jnp.max(s64, axis=-1, keepdims=True)
    ew = jnp.exp(s64 - m)
    denom = jnp.sum(ew, axis=-1, keepdims=True)
    hw = (h * ew * (float(R) / denom)).astype(jnp.bfloat16)

    combined = jnp.dot(hw, bmb[...], preferred_element_type=jnp.float32)
    out_ref[...] = result + b_ref[...] + combined


@jax.jit
def kernel(x, W, b, Wq, Wk, A, Bm):
    rows = B * S
    xf = x.reshape(rows, IN)
    af = A.reshape(E * R, IN)
    b2 = b.reshape(1, OUT)

    grid = (rows // TILE,)
    const = lambda shape: pl.BlockSpec(shape, lambda i: tuple(0 for _ in shape))
    out = pl.pallas_call(
        _kernel,
        grid=grid,
        in_specs=[
            pl.BlockSpec((TILE, IN), lambda i: (i, 0)),
            const((OUT, IN)),
            const((KQ, IN)),
            const((KQ, IN)),
            const((E * R, IN)),
            const((E, OUT, R)),
            const((1, OUT)),
        ],
        out_specs=pl.BlockSpec((TILE, OUT), lambda i: (i, 0)),
        out_shape=jax.ShapeDtypeStruct((rows, OUT), jnp.float32),
        scratch_shapes=[
            pltpu.VMEM((OUT, IN), jnp.bfloat16),
            pltpu.VMEM((KQ, IN), jnp.bfloat16),
            pltpu.VMEM((KQ, IN), jnp.bfloat16),
            pltpu.VMEM((E * R, IN), jnp.bfloat16),
            pltpu.VMEM((E * R, OUT), jnp.bfloat16),
            pltpu.VMEM((KQ, E * R), jnp.bfloat16),
        ],
    )(xf, W, Wq, Wk, af, Bm, b2)
    return out.reshape(B, S, OUT)


# speedup vs baseline: 1.1036x; 1.1036x over previous
import math

import jax
import jax.numpy as jnp
from jax.experimental import pallas as pl

B, S, IN, OUT, E, R, DK = 2, 4096, 768, 768, 8, 8, 32
SCALING = 16.0 / 8.0
TILE = 2048
KQ = E * DK
WIDE = OUT + 2 * KQ + E * R

_NT = (((1,), (1,)), ((), ()))


def _kernel(x_ref, wcat_ref, b_ref, segrep_ref, bmf_ref, out_ref):
    xb = x_ref[...].astype(jnp.bfloat16)

    big = jax.lax.dot_general(xb, wcat_ref[...], _NT,
                              preferred_element_type=jnp.float32)
    result = big[:, :OUT]
    q = big[:, OUT:OUT + KQ]
    k = big[:, OUT + KQ:OUT + 2 * KQ]
    h = big[:, OUT + 2 * KQ:]

    qk = (q * k).astype(jnp.bfloat16)
    s64 = jnp.dot(qk, segrep_ref[...], preferred_element_type=jnp.float32)
    m = jnp.max(s64, axis=-1, keepdims=True)
    ew = jnp.exp(s64 - m)
    denom = jnp.sum(ew, axis=-1, keepdims=True)
    hw = (h * ew * (float(R) / denom)).astype(jnp.bfloat16)

    combined = jnp.dot(hw, bmf_ref[...], preferred_element_type=jnp.float32)
    out_ref[...] = result + b_ref[...] + combined


@jax.jit
def kernel(x, W, b, Wq, Wk, A, Bm):
    rows = B * S
    xf = x.reshape(rows, IN)
    wcat = jnp.concatenate(
        [W, Wq, Wk, A.reshape(E * R, IN)], axis=0).astype(jnp.bfloat16)
    bmf = (jnp.transpose(Bm, (0, 2, 1)).reshape(E * R, OUT)
           * SCALING).astype(jnp.bfloat16)
    b2 = b.reshape(1, OUT)
    j = jnp.arange(KQ)[:, None] // DK
    e = jnp.arange(E * R)[None, :] // R
    segrep = ((j == e).astype(jnp.float32)
              * (1.0 / math.sqrt(DK))).astype(jnp.bfloat16)

    grid = (rows // TILE,)
    out = pl.pallas_call(
        _kernel,
        grid=grid,
        in_specs=[
            pl.BlockSpec((TILE, IN), lambda i: (i, 0)),
            pl.BlockSpec((WIDE, IN), lambda i: (0, 0)),
            pl.BlockSpec((1, OUT), lambda i: (0, 0)),
            pl.BlockSpec((KQ, E * R), lambda i: (0, 0)),
            pl.BlockSpec((E * R, OUT), lambda i: (0, 0)),
        ],
        out_specs=pl.BlockSpec((TILE, OUT), lambda i: (i, 0)),
        out_shape=jax.ShapeDtypeStruct((rows, OUT), jnp.float32),
    )(xf, wcat, b2, segrep, bmf)
    return out.reshape(B, S, OUT)


# speedup vs baseline: 1.1568x; 1.0482x over previous
import math

import jax
import jax.numpy as jnp
from jax.experimental import pallas as pl
from jax.experimental.pallas import tpu as pltpu

B, S, IN, OUT, E, R, DK = 2, 4096, 768, 768, 8, 8, 32
SCALING = 16.0 / 8.0
TILE = 2048
KQ = E * DK
WIDE = OUT + 2 * KQ + E * R

_NT = (((1,), (1,)), ((), ()))


def _kernel(x_ref, w_ref, wq_ref, wk_ref, a_ref, bm_ref, b_ref, out_ref,
            wcat, bmf, seg):
    @pl.when(pl.program_id(0) == 0)
    def _prep():
        wcat[pl.ds(0, OUT), :] = w_ref[...].astype(jnp.bfloat16)
        wcat[pl.ds(OUT, KQ), :] = wq_ref[...].astype(jnp.bfloat16)
        wcat[pl.ds(OUT + KQ, KQ), :] = wk_ref[...].astype(jnp.bfloat16)
        wcat[pl.ds(OUT + 2 * KQ, E * R), :] = a_ref[...].astype(jnp.bfloat16)
        bmt = jax.lax.transpose(bm_ref[...], (0, 2, 1))
        bmf[...] = (bmt.reshape(E * R, OUT) * SCALING).astype(jnp.bfloat16)
        j = jax.lax.broadcasted_iota(jnp.int32, (KQ, E * R), 0) // DK
        e = jax.lax.broadcasted_iota(jnp.int32, (KQ, E * R), 1) // R
        seg[...] = jnp.where(j == e, 1.0 / math.sqrt(DK),
                             0.0).astype(jnp.bfloat16)

    xb = x_ref[...].astype(jnp.bfloat16)

    big = jax.lax.dot_general(xb, wcat[...], _NT,
                              preferred_element_type=jnp.float32)
    result = big[:, :OUT]
    q = big[:, OUT:OUT + KQ]
    k = big[:, OUT + KQ:OUT + 2 * KQ]
    h = big[:, OUT + 2 * KQ:]

    qk = (q * k).astype(jnp.bfloat16)
    s64 = jnp.dot(qk, seg[...], preferred_element_type=jnp.float32)
    m = jnp.max(s64, axis=-1, keepdims=True)
    ew = jnp.exp(s64 - m)
    denom = jnp.sum(ew, axis=-1, keepdims=True)
    hw = (h * ew * (float(R) / denom)).astype(jnp.bfloat16)

    combined = jnp.dot(hw, bmf[...], preferred_element_type=jnp.float32)
    out_ref[...] = result + b_ref[...] + combined


@jax.jit
def kernel(x, W, b, Wq, Wk, A, Bm):
    rows = B * S
    xf = x.reshape(rows, IN)
    af = A.reshape(E * R, IN)
    b2 = b.reshape(1, OUT)

    grid = (rows // TILE,)
    const = lambda shape: pl.BlockSpec(shape, lambda i: tuple(0 for _ in shape))
    out = pl.pallas_call(
        _kernel,
        grid=grid,
        in_specs=[
            pl.BlockSpec((TILE, IN), lambda i: (i, 0)),
            const((OUT, IN)),
            const((KQ, IN)),
            const((KQ, IN)),
            const((E * R, IN)),
            const((E, OUT, R)),
            const((1, OUT)),
        ],
        out_specs=pl.BlockSpec((TILE, OUT), lambda i: (i, 0)),
        out_shape=jax.ShapeDtypeStruct((rows, OUT), jnp.float32),
        scratch_shapes=[
            pltpu.VMEM((WIDE, IN), jnp.bfloat16),
            pltpu.VMEM((E * R, OUT), jnp.bfloat16),
            pltpu.VMEM((KQ, E * R), jnp.bfloat16),
        ],
    )(xf, W, Wq, Wk, af, Bm, b2)
    return out.reshape(B, S, OUT)
